# Initial kernel scaffold; baseline (speedup 1.0000x reference)
import jax
import jax.numpy as jnp
from jax.experimental import pallas as pl


def kernel(x, edge_index, W, b):
    raise NotImplementedError("write your pallas kernel here")


# speedup vs baseline: 18.3773x; 18.3773x over previous
import functools

import jax
import jax.numpy as jnp
from jax import lax
from jax.experimental import pallas as pl
from jax.experimental.pallas import tpu as pltpu
from jax.experimental.pallas import tpu_sc as plsc

N_NODES = 10000
N_EDGES = 320000
D_IN = 128
D_OUT = 64

NC = 2
NS = 16
NW = NC * NS
NPAD = 10240
RPT = NPAD // NS
EPW = N_EDGES // NW
CH = 80
NCHUNK = EPW // CH
DEG_W = 8

_mesh = plsc.VectorSubcoreMesh(core_axis_name="c", subcore_axis_name="s")
_sc_params = pltpu.CompilerParams(use_tc_tiling_on_sc=False)


@functools.partial(
    pl.kernel,
    out_type=jax.ShapeDtypeStruct((NC * NPAD, DEG_W), jnp.float32),
    mesh=_mesh,
    compiler_params=_sc_params,
    scratch_types=[
        pltpu.VMEM_SHARED((NPAD, DEG_W), jnp.float32),
        pltpu.VMEM((CH,), jnp.int32),
        pltpu.VMEM((CH, DEG_W), jnp.float32),
    ],
)
def _deg_kernel(dst_hbm, zeros_hbm, ones_hbm, out_hbm, acc_sh, idx_v, ones_v):
    cid = lax.axis_index("c")
    sid = lax.axis_index("s")
    wid = sid * NC + cid
    pltpu.sync_copy(zeros_hbm, acc_sh.at[pl.ds(sid * RPT, RPT)])
    pltpu.sync_copy(ones_hbm, ones_v)
    plsc.subcore_barrier()

    def body(j, _):
        base = pl.multiple_of(wid * EPW + j * CH, 8)
        pltpu.sync_copy(dst_hbm.at[pl.ds(base, CH)], idx_v)
        pltpu.sync_copy(ones_v, acc_sh.at[idx_v], add=True)
        return 0

    lax.fori_loop(0, NCHUNK, body, 0)
    plsc.subcore_barrier()
    pltpu.sync_copy(
        acc_sh.at[pl.ds(sid * RPT, RPT)],
        out_hbm.at[pl.ds(cid * NPAD + sid * RPT, RPT)],
    )


@functools.partial(
    pl.kernel,
    out_type=jax.ShapeDtypeStruct((NC * NPAD, D_OUT), jnp.float32),
    mesh=_mesh,
    compiler_params=_sc_params,
    scratch_types=[
        pltpu.VMEM_SHARED((NPAD, D_OUT), jnp.float32),
        pltpu.VMEM((CH,), jnp.int32),
        pltpu.VMEM((CH,), jnp.int32),
        pltpu.VMEM((CH, D_OUT), jnp.float32),
        pltpu.SemaphoreType.DMA,
    ],
)
def _scatter_kernel(hs_hbm, src_hbm, dst_hbm, zeros_hbm, out_hbm,
                    acc_sh, src_v, dst_v, rows_v, sem):
    cid = lax.axis_index("c")
    sid = lax.axis_index("s")
    wid = sid * NC + cid
    pltpu.sync_copy(zeros_hbm, acc_sh.at[pl.ds(sid * RPT, RPT)])
    plsc.subcore_barrier()

    def body(j, _):
        base = pl.multiple_of(wid * EPW + j * CH, 8)
        pltpu.sync_copy(src_hbm.at[pl.ds(base, CH)], src_v)
        pltpu.sync_copy(dst_hbm.at[pl.ds(base, CH)], dst_v)
        pltpu.async_copy(hs_hbm.at[src_v], rows_v, sem).wait()
        pltpu.sync_copy(rows_v, acc_sh.at[dst_v], add=True)
        return 0

    lax.fori_loop(0, NCHUNK, body, 0)
    plsc.subcore_barrier()
    pltpu.sync_copy(
        acc_sh.at[pl.ds(sid * RPT, RPT)],
        out_hbm.at[pl.ds(cid * NPAD + sid * RPT, RPT)],
    )


_BN = 1000


def _linear_body(x_ref, w_ref, d0_ref, d1_ref, hs_ref, dis_ref):
    deg = d0_ref[...] + d1_ref[...] + 1.0
    dis = lax.rsqrt(deg)
    h = jnp.dot(x_ref[...], w_ref[...], preferred_element_type=jnp.float32)
    hs_ref[...] = h * dis
    dis_ref[...] = dis


def _linear(x, W, d0, d1):
    grid = N_NODES // _BN
    return pl.pallas_call(
        _linear_body,
        grid=(grid,),
        in_specs=[
            pl.BlockSpec((_BN, D_IN), lambda i: (i, 0)),
            pl.BlockSpec((D_IN, D_OUT), lambda i: (0, 0)),
            pl.BlockSpec((_BN, 1), lambda i: (i, 0)),
            pl.BlockSpec((_BN, 1), lambda i: (i, 0)),
        ],
        out_specs=[
            pl.BlockSpec((_BN, D_OUT), lambda i: (i, 0)),
            pl.BlockSpec((_BN, 1), lambda i: (i, 0)),
        ],
        out_shape=[
            jax.ShapeDtypeStruct((N_NODES, D_OUT), jnp.float32),
            jax.ShapeDtypeStruct((N_NODES, 1), jnp.float32),
        ],
    )(x, W, d0, d1)


def _combine_body(s0_ref, s1_ref, hs_ref, dis_ref, b_ref, out_ref):
    out_ref[...] = (
        dis_ref[...] * (s0_ref[...] + s1_ref[...] + hs_ref[...]) + b_ref[...]
    )


def _combine(s0, s1, hs, dis, b2):
    grid = N_NODES // _BN
    return pl.pallas_call(
        _combine_body,
        grid=(grid,),
        in_specs=[
            pl.BlockSpec((_BN, D_OUT), lambda i: (i, 0)),
            pl.BlockSpec((_BN, D_OUT), lambda i: (i, 0)),
            pl.BlockSpec((_BN, D_OUT), lambda i: (i, 0)),
            pl.BlockSpec((_BN, 1), lambda i: (i, 0)),
            pl.BlockSpec((1, D_OUT), lambda i: (0, 0)),
        ],
        out_specs=pl.BlockSpec((_BN, D_OUT), lambda i: (i, 0)),
        out_shape=jax.ShapeDtypeStruct((N_NODES, D_OUT), jnp.float32),
    )(s0, s1, hs, dis, b2)


def kernel(x, edge_index, W, b):
    src = edge_index[0]
    dst = edge_index[1]
    z_deg = jnp.zeros((RPT, DEG_W), jnp.float32)
    ones = jnp.ones((CH, DEG_W), jnp.float32)
    z_acc = jnp.zeros((RPT, D_OUT), jnp.float32)

    deg_parts = _deg_kernel(dst, z_deg, ones)
    d0 = deg_parts[:N_NODES, 0:1]
    d1 = deg_parts[NPAD:NPAD + N_NODES, 0:1]

    hs, dis = _linear(x, W, d0, d1)

    s_parts = _scatter_kernel(hs, src, dst, z_acc)
    s0 = s_parts[:N_NODES]
    s1 = s_parts[NPAD:NPAD + N_NODES]

    return _combine(s0, s1, hs, dis, jnp.reshape(b, (1, D_OUT)))


# speedup vs baseline: 41.4353x; 2.2547x over previous
import functools

import jax
import jax.numpy as jnp
from jax import lax
from jax.experimental import pallas as pl
from jax.experimental.pallas import tpu as pltpu
from jax.experimental.pallas import tpu_sc as plsc

N_NODES = 10000
N_EDGES = 320000
D_IN = 128
D_OUT = 64

NC = 2
NS = 16
NW = NC * NS
NPAD = 10240
RPT = NPAD // NS
CH = 128
NCH = 80
EPW = NCH * CH
E_PAD = NW * EPW
DEG_W = 8

_mesh = plsc.VectorSubcoreMesh(core_axis_name="c", subcore_axis_name="s")
_sc_params = pltpu.CompilerParams(use_tc_tiling_on_sc=False)


def _make_deg_kernel(deg_w, mode):
    @functools.partial(
        pl.kernel,
        out_type=jax.ShapeDtypeStruct((NC * NPAD, deg_w), jnp.float32),
        mesh=_mesh,
        compiler_params=_sc_params,
        scratch_types=[
            pltpu.VMEM_SHARED((NPAD, deg_w), jnp.float32),
            pltpu.VMEM((NCH, CH), jnp.int32),
            pltpu.VMEM((CH, deg_w), jnp.float32),
            pltpu.SemaphoreType.DMA,
        ],
    )
    def _deg_kernel(dst_hbm, zeros_hbm, ones_hbm, out_hbm,
                    acc_sh, dst_all, ones_v, sem):
        cid = lax.axis_index("c")
        sid = lax.axis_index("s")
        wid = sid * NC + cid
        pltpu.sync_copy(zeros_hbm, acc_sh.at[pl.ds(sid * RPT, RPT)])
        pltpu.sync_copy(ones_hbm, ones_v)
        pltpu.sync_copy(dst_hbm.at[wid], dst_all)
        plsc.subcore_barrier()

        if mode == "sync":
            def body(j, _):
                pltpu.sync_copy(ones_v, acc_sh.at[dst_all.at[j]], add=True)
                return 0

            lax.fori_loop(0, NCH, body, 0)
        else:
            GRP = 8
            NG = NCH // GRP

            def fire(g):
                for i in range(GRP):
                    pltpu.async_copy(ones_v, acc_sh.at[dst_all.at[g * GRP + i]],
                                     sem, add=True)

            def drain(g):
                for i in range(GRP):
                    pltpu.make_async_copy(
                        ones_v, acc_sh.at[dst_all.at[g * GRP + i]], sem).wait()

            fire(0)

            def body(g, _):
                fire(g)
                drain(g - 1)
                return 0

            lax.fori_loop(1, NG, body, 0)
            drain(NG - 1)
        plsc.subcore_barrier()
        pltpu.sync_copy(
            acc_sh.at[pl.ds(sid * RPT, RPT)],
            out_hbm.at[pl.ds(cid * NPAD + sid * RPT, RPT)],
        )

    return _deg_kernel


_deg_kernel = _make_deg_kernel(DEG_W, "async")


@functools.partial(
    pl.kernel,
    out_type=jax.ShapeDtypeStruct((NC * NPAD, D_OUT), jnp.float32),
    mesh=_mesh,
    compiler_params=_sc_params,
    scratch_types=[
        pltpu.VMEM_SHARED((NPAD, D_OUT), jnp.float32),
        pltpu.VMEM((NCH, CH), jnp.int32),
        pltpu.VMEM((NCH, CH), jnp.int32),
        pltpu.VMEM((CH, D_OUT), jnp.float32),
        pltpu.VMEM((CH, D_OUT), jnp.float32),
        pltpu.SemaphoreType.DMA,
        pltpu.SemaphoreType.DMA,
    ],
)
def _scatter_kernel(hs_hbm, src_hbm, dst_hbm, zeros_hbm, out_hbm,
                    acc_sh, src_all, dst_all, rows_a, rows_b, sem_a, sem_b):
    cid = lax.axis_index("c")
    sid = lax.axis_index("s")
    wid = sid * NC + cid
    row0 = pl.ds(sid * RPT, RPT)

    @pl.when(cid == 0)
    def _():
        pltpu.sync_copy(hs_hbm.at[row0], acc_sh.at[row0])

    @pl.when(cid == 1)
    def _():
        pltpu.sync_copy(zeros_hbm, acc_sh.at[row0])

    pltpu.sync_copy(src_hbm.at[wid], src_all)
    pltpu.sync_copy(dst_hbm.at[wid], dst_all)
    plsc.subcore_barrier()

    def gather(j, buf, sem):
        pltpu.async_copy(hs_hbm.at[src_all.at[j]], buf, sem)

    def gather_wait(j, buf, sem):
        pltpu.make_async_copy(hs_hbm.at[src_all.at[j]], buf, sem).wait()

    def scat(j, buf):
        pltpu.sync_copy(buf, acc_sh.at[dst_all.at[j]], add=True)

    gather(0, rows_a, sem_a)
    gather(1, rows_b, sem_b)

    def body(k, _):
        c0 = k * 2
        gather_wait(c0, rows_a, sem_a)
        scat(c0, rows_a)
        gather(c0 + 2, rows_a, sem_a)
        c1 = c0 + 1
        gather_wait(c1, rows_b, sem_b)
        scat(c1, rows_b)
        gather(c1 + 2, rows_b, sem_b)
        return 0

    lax.fori_loop(0, NCH // 2 - 1, body, 0)
    gather_wait(NCH - 2, rows_a, sem_a)
    scat(NCH - 2, rows_a)
    gather_wait(NCH - 1, rows_b, sem_b)
    scat(NCH - 1, rows_b)
    plsc.subcore_barrier()
    pltpu.sync_copy(
        acc_sh.at[row0],
        out_hbm.at[pl.ds(cid * NPAD + sid * RPT, RPT)],
    )


_BN = NPAD // 16


def _linear_body(x_ref, w_ref, d0_ref, d1_ref, hs_ref, dis_ref):
    deg = d0_ref[...] + d1_ref[...] + 1.0
    dis = lax.rsqrt(deg)
    h = jnp.dot(x_ref[...], w_ref[...], preferred_element_type=jnp.float32)
    hs_ref[...] = h * dis
    dis_ref[...] = dis


def _linear(x, W, d0, d1):
    return pl.pallas_call(
        _linear_body,
        grid=(NPAD // _BN,),
        in_specs=[
            pl.BlockSpec((_BN, D_IN), lambda i: (i, 0)),
            pl.BlockSpec((D_IN, D_OUT), lambda i: (0, 0)),
            pl.BlockSpec((_BN, 1), lambda i: (i, 0)),
            pl.BlockSpec((_BN, 1), lambda i: (i, 0)),
        ],
        out_specs=[
            pl.BlockSpec((_BN, D_OUT), lambda i: (i, 0)),
            pl.BlockSpec((_BN, 1), lambda i: (i, 0)),
        ],
        out_shape=[
            jax.ShapeDtypeStruct((NPAD, D_OUT), jnp.float32),
            jax.ShapeDtypeStruct((NPAD, 1), jnp.float32),
        ],
    )(x, W, d0, d1)


def _combine_body(s0_ref, s1_ref, dis_ref, b_ref, out_ref):
    out_ref[...] = dis_ref[...] * (s0_ref[...] + s1_ref[...]) + b_ref[...]


def _combine(s0, s1, dis, b2):
    return pl.pallas_call(
        _combine_body,
        grid=(NPAD // _BN,),
        in_specs=[
            pl.BlockSpec((_BN, D_OUT), lambda i: (i, 0)),
            pl.BlockSpec((_BN, D_OUT), lambda i: (i, 0)),
            pl.BlockSpec((_BN, 1), lambda i: (i, 0)),
            pl.BlockSpec((1, D_OUT), lambda i: (0, 0)),
        ],
        out_specs=pl.BlockSpec((_BN, D_OUT), lambda i: (i, 0)),
        out_shape=jax.ShapeDtypeStruct((NPAD, D_OUT), jnp.float32),
    )(s0, s1, dis, b2)


def kernel(x, edge_index, W, b):
    pad = (jnp.arange(E_PAD - N_EDGES, dtype=jnp.int32) % (NPAD - N_NODES)
           + N_NODES)
    src = jnp.reshape(jnp.concatenate([edge_index[0], pad]), (NW, NCH, CH))
    dst = jnp.reshape(jnp.concatenate([edge_index[1], pad]), (NW, NCH, CH))
    x_pad = jnp.pad(x, ((0, NPAD - N_NODES), (0, 0)))
    z_deg = jnp.zeros((RPT, DEG_W), jnp.float32)
    ones = jnp.ones((CH, DEG_W), jnp.float32)

    deg_parts = _deg_kernel(dst, z_deg, ones)
    d0 = deg_parts[:NPAD, 0:1]
    d1 = deg_parts[NPAD:, 0:1]

    hs, dis = _linear(x_pad, W, d0, d1)

    z_acc = jnp.zeros((RPT, D_OUT), jnp.float32)
    s_parts = _scatter_kernel(hs, src, dst, z_acc)
    s0 = s_parts[:NPAD]
    s1 = s_parts[NPAD:]

    out = _combine(s0, s1, dis, jnp.reshape(b, (1, D_OUT)))
    return out[:N_NODES]


# speedup vs baseline: 46.4587x; 1.1212x over previous
import functools

import jax
import jax.numpy as jnp
from jax import lax
from jax.experimental import pallas as pl
from jax.experimental.pallas import tpu as pltpu
from jax.experimental.pallas import tpu_sc as plsc

N_NODES = 10000
D_IN = 128
D_OUT = 64

NC = 2
NS = 16
NW = NC * NS
NPAD = 10240
RPT = NPAD // NS
LASTR = N_NODES - 15 * RPT
CH = 80
NCH = 125
DEG_W = 8

_mesh = plsc.VectorSubcoreMesh(core_axis_name="c", subcore_axis_name="s")
_sc_params = pltpu.CompilerParams(use_tc_tiling_on_sc=False)


@functools.partial(
    pl.kernel,
    out_type=[jax.ShapeDtypeStruct((NPAD, DEG_W), jnp.float32)] * NC,
    mesh=_mesh,
    compiler_params=_sc_params,
    scratch_types=[
        pltpu.VMEM_SHARED((NPAD, DEG_W), jnp.float32),
        pltpu.VMEM((NCH, CH), jnp.int32),
        pltpu.VMEM((CH, DEG_W), jnp.float32),
        pltpu.SemaphoreType.DMA,
    ],
)
def _deg_kernel(dst_hbm, zeros_hbm, ones_hbm, out0_hbm, out1_hbm,
                acc_sh, dst_all, ones_v, sem):
    cid = lax.axis_index("c")
    sid = lax.axis_index("s")
    wid = sid * NC + cid
    row0 = pl.ds(sid * RPT, RPT)
    pltpu.sync_copy(zeros_hbm, acc_sh.at[row0])
    pltpu.sync_copy(ones_hbm, ones_v)
    pltpu.sync_copy(dst_hbm.at[wid], dst_all)
    plsc.subcore_barrier()

    GRP = 5
    NG = NCH // GRP

    def fire(g):
        for i in range(GRP):
            pltpu.async_copy(ones_v, acc_sh.at[dst_all.at[g * GRP + i]],
                             sem, add=True)

    def drain(g):
        for i in range(GRP):
            pltpu.make_async_copy(
                ones_v, acc_sh.at[dst_all.at[g * GRP + i]], sem).wait()

    fire(0)

    def body(g, _):
        fire(g)
        drain(g - 1)
        return 0

    lax.fori_loop(1, NG, body, 0)
    drain(NG - 1)
    plsc.subcore_barrier()

    @pl.when(cid == 0)
    def _():
        pltpu.sync_copy(acc_sh.at[row0], out0_hbm.at[row0])

    @pl.when(cid == 1)
    def _():
        pltpu.sync_copy(acc_sh.at[row0], out1_hbm.at[row0])


@functools.partial(
    pl.kernel,
    out_type=[jax.ShapeDtypeStruct((NPAD, D_OUT), jnp.float32)] * NC,
    mesh=_mesh,
    compiler_params=_sc_params,
    scratch_types=[
        pltpu.VMEM_SHARED((NPAD, D_OUT), jnp.float32),
        pltpu.VMEM((NCH, CH), jnp.int32),
        pltpu.VMEM((NCH, CH), jnp.int32),
        [pltpu.VMEM((CH, D_OUT), jnp.float32)] * 4,
        [pltpu.SemaphoreType.DMA] * 4,
        [pltpu.SemaphoreType.DMA] * 4,
    ],
)
def _scatter_kernel(hs_hbm, src_hbm, dst_hbm, zeros_hbm, out0_hbm, out1_hbm,
                    acc_sh, src_all, dst_all, rows, semg, sems):
    cid = lax.axis_index("c")
    sid = lax.axis_index("s")
    wid = sid * NC + cid
    row0 = pl.ds(sid * RPT, RPT)
    rowl = pl.ds(15 * RPT, LASTR)

    @pl.when(jnp.logical_and(cid == 0, sid < 15))
    def _():
        pltpu.sync_copy(hs_hbm.at[row0], acc_sh.at[row0])

    @pl.when(jnp.logical_and(cid == 0, sid == 15))
    def _():
        pltpu.sync_copy(hs_hbm.at[rowl], acc_sh.at[rowl])

    @pl.when(jnp.logical_and(cid == 1, sid < 15))
    def _():
        pltpu.sync_copy(zeros_hbm, acc_sh.at[row0])

    @pl.when(jnp.logical_and(cid == 1, sid == 15))
    def _():
        pltpu.sync_copy(zeros_hbm.at[pl.ds(0, LASTR)], acc_sh.at[rowl])

    pltpu.sync_copy(src_hbm.at[wid], src_all)
    pltpu.sync_copy(dst_hbm.at[wid], dst_all)
    plsc.subcore_barrier()

    def gather(j, b):
        pltpu.async_copy(hs_hbm.at[src_all.at[j]], rows[b], semg[b])

    def gather_wait(j, b):
        pltpu.make_async_copy(hs_hbm.at[src_all.at[j]], rows[b], semg[b]).wait()

    def scat(j, b):
        pltpu.async_copy(rows[b], acc_sh.at[dst_all.at[j]], sems[b], add=True)

    def scat_wait(j, b):
        pltpu.make_async_copy(rows[b], acc_sh.at[dst_all.at[j]], sems[b]).wait()

    for b in range(4):
        gather(b, b)

    def body(k, _):
        for i in range(4):
            c = k * 4 + i
            b = i
            gather_wait(c, b)
            scat(c, b)
            br = (i + 2) % 4

            @pl.when(jnp.logical_and(c >= 2, c <= NCH - 3))
            def _(c=c, b=br):
                scat_wait(c - 2, b)
                gather(c + 2, b)

        return 0

    lax.fori_loop(0, NCH // 4, body, 0)
    gather_wait(NCH - 1, 0)
    scat(NCH - 1, 0)
    scat_wait(NCH - 4, 1)
    scat_wait(NCH - 3, 2)
    scat_wait(NCH - 2, 3)
    scat_wait(NCH - 1, 0)
    plsc.subcore_barrier()

    @pl.when(cid == 0)
    def _():
        pltpu.sync_copy(acc_sh.at[row0], out0_hbm.at[row0])

    @pl.when(cid == 1)
    def _():
        pltpu.sync_copy(acc_sh.at[row0], out1_hbm.at[row0])


_BN = 1000


def _linear_body(x_ref, w_ref, d0_ref, d1_ref, hs_ref, dis_ref):
    deg = d0_ref[...][:, 0:1] + d1_ref[...][:, 0:1] + 1.0
    dis = lax.rsqrt(deg)
    h = jnp.dot(x_ref[...], w_ref[...], preferred_element_type=jnp.float32)
    hs_ref[...] = h * dis
    dis_ref[...] = dis


def _linear(x, W, d0, d1):
    return pl.pallas_call(
        _linear_body,
        grid=(N_NODES // _BN,),
        in_specs=[
            pl.BlockSpec((_BN, D_IN), lambda i: (i, 0)),
            pl.BlockSpec((D_IN, D_OUT), lambda i: (0, 0)),
            pl.BlockSpec((_BN, DEG_W), lambda i: (i, 0)),
            pl.BlockSpec((_BN, DEG_W), lambda i: (i, 0)),
        ],
        out_specs=[
            pl.BlockSpec((_BN, D_OUT), lambda i: (i, 0)),
            pl.BlockSpec((_BN, 1), lambda i: (i, 0)),
        ],
        out_shape=[
            jax.ShapeDtypeStruct((N_NODES, D_OUT), jnp.float32),
            jax.ShapeDtypeStruct((N_NODES, 1), jnp.float32),
        ],
    )(x, W, d0, d1)


def _combine_body(s0_ref, s1_ref, dis_ref, b_ref, out_ref):
    out_ref[...] = dis_ref[...] * (s0_ref[...] + s1_ref[...]) + b_ref[...]


def _combine(s0, s1, dis, b2):
    return pl.pallas_call(
        _combine_body,
        grid=(N_NODES // _BN,),
        in_specs=[
            pl.BlockSpec((_BN, D_OUT), lambda i: (i, 0)),
            pl.BlockSpec((_BN, D_OUT), lambda i: (i, 0)),
            pl.BlockSpec((_BN, 1), lambda i: (i, 0)),
            pl.BlockSpec((1, D_OUT), lambda i: (0, 0)),
        ],
        out_specs=pl.BlockSpec((_BN, D_OUT), lambda i: (i, 0)),
        out_shape=jax.ShapeDtypeStruct((N_NODES, D_OUT), jnp.float32),
    )(s0, s1, dis, b2)


def kernel(x, edge_index, W, b):
    src = jnp.reshape(edge_index[0], (NW, NCH, CH))
    dst = jnp.reshape(edge_index[1], (NW, NCH, CH))
    z_deg = jnp.zeros((RPT, DEG_W), jnp.float32)
    ones = jnp.ones((CH, DEG_W), jnp.float32)
    z_acc = jnp.zeros((RPT, D_OUT), jnp.float32)

    d0, d1 = _deg_kernel(dst, z_deg, ones)
    hs, dis = _linear(x, W, d0, d1)
    s0, s1 = _scatter_kernel(hs, src, dst, z_acc)
    return _combine(s0, s1, dis, jnp.reshape(b, (1, D_OUT)))


# speedup vs baseline: 48.8687x; 1.0519x over previous
import functools

import jax
import jax.numpy as jnp
from jax import lax
from jax.experimental import pallas as pl
from jax.experimental.pallas import tpu as pltpu
from jax.experimental.pallas import tpu_sc as plsc

N_NODES = 10000
D_IN = 128
D_OUT = 64

NC = 2
NS = 16
NW = NC * NS
NPAD = 10240
RPT = NPAD // NS
LASTR = N_NODES - 15 * RPT
CH = 80
NCH = 125
DEG_W = 8

_mesh = plsc.VectorSubcoreMesh(core_axis_name="c", subcore_axis_name="s")
_sc_params = pltpu.CompilerParams(use_tc_tiling_on_sc=False)


@functools.partial(
    pl.kernel,
    out_type=[jax.ShapeDtypeStruct((NPAD, DEG_W), jnp.float32)] * NC,
    mesh=_mesh,
    compiler_params=_sc_params,
    scratch_types=[
        pltpu.VMEM_SHARED((NPAD, DEG_W), jnp.float32),
        pltpu.VMEM((NCH * CH,), jnp.int32),
        pltpu.VMEM((CH, DEG_W), jnp.float32),
        pltpu.SemaphoreType.DMA,
    ],
)
def _deg_kernel(dst_hbm, zeros_hbm, ones_hbm, out0_hbm, out1_hbm,
                acc_sh, dst_all, ones_v, sem):
    cid = lax.axis_index("c")
    sid = lax.axis_index("s")
    wid = sid * NC + cid
    row0 = pl.ds(sid * RPT, RPT)
    pltpu.sync_copy(zeros_hbm, acc_sh.at[row0])
    pltpu.sync_copy(ones_hbm, ones_v)
    pltpu.sync_copy(dst_hbm.at[pl.ds(wid * (NCH * CH), NCH * CH)], dst_all)
    plsc.subcore_barrier()

    GRP = 5
    NG = NCH // GRP

    def idx(j):
        return dst_all.at[pl.ds(j * CH, CH)]

    def fire(g):
        for i in range(GRP):
            pltpu.async_copy(ones_v, acc_sh.at[idx(g * GRP + i)], sem, add=True)

    def drain(g):
        for i in range(GRP):
            pltpu.make_async_copy(ones_v, acc_sh.at[idx(g * GRP + i)],
                                  sem).wait()

    fire(0)

    def body(g, _):
        fire(g)
        drain(g - 1)
        return 0

    lax.fori_loop(1, NG, body, 0)
    drain(NG - 1)
    plsc.subcore_barrier()

    @pl.when(cid == 0)
    def _():
        pltpu.sync_copy(acc_sh.at[row0], out0_hbm.at[row0])

    @pl.when(cid == 1)
    def _():
        pltpu.sync_copy(acc_sh.at[row0], out1_hbm.at[row0])


@functools.partial(
    pl.kernel,
    out_type=[jax.ShapeDtypeStruct((NPAD, D_OUT), jnp.float32)] * NC,
    mesh=_mesh,
    compiler_params=_sc_params,
    scratch_types=[
        pltpu.VMEM_SHARED((NPAD, D_OUT), jnp.float32),
        pltpu.VMEM_SHARED((NPAD, D_OUT), jnp.float32),
        pltpu.VMEM((NCH * CH,), jnp.int32),
        pltpu.VMEM((NCH * CH,), jnp.int32),
        [pltpu.VMEM((CH, D_OUT), jnp.float32)] * 4,
        [pltpu.SemaphoreType.DMA] * 4,
        [pltpu.SemaphoreType.DMA] * 4,
    ],
)
def _scatter_kernel(hs_hbm, src_hbm, dst_hbm, zeros_hbm, out0_hbm, out1_hbm,
                    acc_sh, hs_sh, src_all, dst_all, rows, semg, sems):
    cid = lax.axis_index("c")
    sid = lax.axis_index("s")
    wid = sid * NC + cid
    row0 = pl.ds(sid * RPT, RPT)
    rowl = pl.ds(15 * RPT, LASTR)

    @pl.when(sid < 15)
    def _():
        pltpu.sync_copy(hs_hbm.at[row0], hs_sh.at[row0])

    @pl.when(sid == 15)
    def _():
        pltpu.sync_copy(hs_hbm.at[rowl], hs_sh.at[rowl])

    @pl.when(jnp.logical_and(cid == 0, sid < 15))
    def _():
        pltpu.sync_copy(hs_hbm.at[row0], acc_sh.at[row0])

    @pl.when(jnp.logical_and(cid == 0, sid == 15))
    def _():
        pltpu.sync_copy(hs_hbm.at[rowl], acc_sh.at[rowl])

    @pl.when(jnp.logical_and(cid == 1, sid < 15))
    def _():
        pltpu.sync_copy(zeros_hbm, acc_sh.at[row0])

    @pl.when(jnp.logical_and(cid == 1, sid == 15))
    def _():
        pltpu.sync_copy(zeros_hbm.at[pl.ds(0, LASTR)], acc_sh.at[rowl])

    pltpu.sync_copy(src_hbm.at[pl.ds(wid * (NCH * CH), NCH * CH)], src_all)
    pltpu.sync_copy(dst_hbm.at[pl.ds(wid * (NCH * CH), NCH * CH)], dst_all)
    plsc.subcore_barrier()

    def sidx(j):
        return src_all.at[pl.ds(j * CH, CH)]

    def didx(j):
        return dst_all.at[pl.ds(j * CH, CH)]

    def gather(j, b):
        pltpu.async_copy(hs_sh.at[sidx(j)], rows[b], semg[b])

    def gather_wait(j, b):
        pltpu.make_async_copy(hs_sh.at[sidx(j)], rows[b], semg[b]).wait()

    def scat(j, b):
        pltpu.async_copy(rows[b], acc_sh.at[didx(j)], sems[b], add=True)

    def scat_wait(j, b):
        pltpu.make_async_copy(rows[b], acc_sh.at[didx(j)], sems[b]).wait()

    for b in range(4):
        gather(b, b)

    def body(k, _):
        for i in range(4):
            c = k * 4 + i
            b = i
            gather_wait(c, b)
            scat(c, b)
            br = (i + 2) % 4

            @pl.when(jnp.logical_and(c >= 2, c <= NCH - 3))
            def _(c=c, b=br):
                scat_wait(c - 2, b)
                gather(c + 2, b)

        return 0

    lax.fori_loop(0, NCH // 4, body, 0)
    gather_wait(NCH - 1, 0)
    scat(NCH - 1, 0)
    scat_wait(NCH - 4, 1)
    scat_wait(NCH - 3, 2)
    scat_wait(NCH - 2, 3)
    scat_wait(NCH - 1, 0)
    plsc.subcore_barrier()

    @pl.when(cid == 0)
    def _():
        pltpu.sync_copy(acc_sh.at[row0], out0_hbm.at[row0])

    @pl.when(cid == 1)
    def _():
        pltpu.sync_copy(acc_sh.at[row0], out1_hbm.at[row0])


_BN = 1000


def _linear_body(x_ref, w_ref, d0_ref, d1_ref, hs_ref, dis_ref):
    deg = d0_ref[...][:, 0:1] + d1_ref[...][:, 0:1] + 1.0
    dis = lax.rsqrt(deg)
    h = jnp.dot(x_ref[...], w_ref[...], preferred_element_type=jnp.float32)
    hs_ref[...] = h * dis
    dis_ref[...] = dis


def _linear(x, W, d0, d1):
    return pl.pallas_call(
        _linear_body,
        grid=(N_NODES // _BN,),
        in_specs=[
            pl.BlockSpec((_BN, D_IN), lambda i: (i, 0)),
            pl.BlockSpec((D_IN, D_OUT), lambda i: (0, 0)),
            pl.BlockSpec((_BN, DEG_W), lambda i: (i, 0)),
            pl.BlockSpec((_BN, DEG_W), lambda i: (i, 0)),
        ],
        out_specs=[
            pl.BlockSpec((_BN, D_OUT), lambda i: (i, 0)),
            pl.BlockSpec((_BN, 1), lambda i: (i, 0)),
        ],
        out_shape=[
            jax.ShapeDtypeStruct((N_NODES, D_OUT), jnp.float32),
            jax.ShapeDtypeStruct((N_NODES, 1), jnp.float32),
        ],
    )(x, W, d0, d1)


def _combine_body(s0_ref, s1_ref, dis_ref, b_ref, out_ref):
    out_ref[...] = dis_ref[...] * (s0_ref[...] + s1_ref[...]) + b_ref[...]


def _combine(s0, s1, dis, b2):
    return pl.pallas_call(
        _combine_body,
        grid=(N_NODES // _BN,),
        in_specs=[
            pl.BlockSpec((_BN, D_OUT), lambda i: (i, 0)),
            pl.BlockSpec((_BN, D_OUT), lambda i: (i, 0)),
            pl.BlockSpec((_BN, 1), lambda i: (i, 0)),
            pl.BlockSpec((1, D_OUT), lambda i: (0, 0)),
        ],
        out_specs=pl.BlockSpec((_BN, D_OUT), lambda i: (i, 0)),
        out_shape=jax.ShapeDtypeStruct((N_NODES, D_OUT), jnp.float32),
    )(s0, s1, dis, b2)


def kernel(x, edge_index, W, b):
    src = edge_index[0]
    dst = edge_index[1]
    z_deg = jnp.zeros((RPT, DEG_W), jnp.float32)
    ones = jnp.ones((CH, DEG_W), jnp.float32)
    z_acc = jnp.zeros((RPT, D_OUT), jnp.float32)

    d0, d1 = _deg_kernel(dst, z_deg, ones)
    hs, dis = _linear(x, W, d0, d1)
    s0, s1 = _scatter_kernel(hs, src, dst, z_acc)
    return _combine(s0, s1, dis, jnp.reshape(b, (1, D_OUT)))


# speedup vs baseline: 54.3249x; 1.1117x over previous
import functools

import jax
import jax.numpy as jnp
from jax import lax
from jax.experimental import pallas as pl
from jax.experimental.pallas import tpu as pltpu
from jax.experimental.pallas import tpu_sc as plsc

N_NODES = 10000
D_IN = 128
D_OUT = 64

NC = 2
NS = 16
NW = NC * NS
NPAD = 10240
RPT = NPAD // NS
LASTR = N_NODES - 15 * RPT
CH = 200
NCH = 50
N_EDGES = NW * NCH * CH
DEG_W = 8

_mesh = plsc.VectorSubcoreMesh(core_axis_name="c", subcore_axis_name="s")
_sc_params = pltpu.CompilerParams(use_tc_tiling_on_sc=False)


@functools.partial(
    pl.kernel,
    out_type=[jax.ShapeDtypeStruct((NPAD, DEG_W), jnp.float32)] * NC,
    mesh=_mesh,
    compiler_params=_sc_params,
    scratch_types=[
        pltpu.VMEM_SHARED((NPAD, DEG_W), jnp.float32),
        pltpu.VMEM((NCH * CH,), jnp.int32),
        pltpu.VMEM((CH, DEG_W), jnp.float32),
        pltpu.SemaphoreType.DMA,
    ],
)
def _deg_kernel(dst_hbm, zeros_hbm, ones_hbm, out0_hbm, out1_hbm,
                acc_sh, dst_all, ones_v, sem):
    cid = lax.axis_index("c")
    sid = lax.axis_index("s")
    wid = sid * NC + cid
    row0 = pl.ds(sid * RPT, RPT)
    pltpu.sync_copy(zeros_hbm, acc_sh.at[row0])
    pltpu.sync_copy(ones_hbm, ones_v)
    pltpu.sync_copy(dst_hbm.at[pl.ds(wid * (NCH * CH), NCH * CH)], dst_all)
    plsc.subcore_barrier()

    GRP = 5
    NG = NCH // GRP

    def idx(j):
        return dst_all.at[pl.ds(j * CH, CH)]

    def fire(g):
        for i in range(GRP):
            pltpu.async_copy(ones_v, acc_sh.at[idx(g * GRP + i)], sem, add=True)

    def drain(g):
        for i in range(GRP):
            pltpu.make_async_copy(ones_v, acc_sh.at[idx(g * GRP + i)],
                                  sem).wait()

    fire(0)

    def body(g, _):
        fire(g)
        drain(g - 1)
        return 0

    lax.fori_loop(1, NG, body, 0)
    drain(NG - 1)
    plsc.subcore_barrier()

    @pl.when(cid == 0)
    def _():
        pltpu.sync_copy(acc_sh.at[row0], out0_hbm.at[row0])

    @pl.when(cid == 1)
    def _():
        pltpu.sync_copy(acc_sh.at[row0], out1_hbm.at[row0])


@functools.partial(
    pl.kernel,
    out_type=[jax.ShapeDtypeStruct((NPAD, D_OUT), jnp.float32)] * NC,
    mesh=_mesh,
    compiler_params=_sc_params,
    scratch_types=[
        pltpu.VMEM_SHARED((NPAD, D_OUT), jnp.float32),
        pltpu.VMEM((NCH * CH,), jnp.int32),
        pltpu.VMEM((NCH * CH,), jnp.int32),
        [pltpu.VMEM((CH, D_OUT), jnp.float32)] * 4,
        [pltpu.SemaphoreType.DMA] * 4,
        [pltpu.SemaphoreType.DMA] * 4,
    ],
)
def _scatter_kernel(hs_hbm, src_hbm, dst_hbm, zeros_hbm, out0_hbm, out1_hbm,
                    acc_sh, src_all, dst_all, rows, semg, sems):
    cid = lax.axis_index("c")
    sid = lax.axis_index("s")
    wid = sid * NC + cid
    row0 = pl.ds(sid * RPT, RPT)
    rowl = pl.ds(15 * RPT, LASTR)

    @pl.when(jnp.logical_and(cid == 0, sid < 15))
    def _():
        pltpu.sync_copy(hs_hbm.at[row0], acc_sh.at[row0])

    @pl.when(jnp.logical_and(cid == 0, sid == 15))
    def _():
        pltpu.sync_copy(hs_hbm.at[rowl], acc_sh.at[rowl])

    @pl.when(jnp.logical_and(cid == 1, sid < 15))
    def _():
        pltpu.sync_copy(zeros_hbm, acc_sh.at[row0])

    @pl.when(jnp.logical_and(cid == 1, sid == 15))
    def _():
        pltpu.sync_copy(zeros_hbm.at[pl.ds(0, LASTR)], acc_sh.at[rowl])

    pltpu.sync_copy(src_hbm.at[pl.ds(wid * (NCH * CH), NCH * CH)], src_all)
    pltpu.sync_copy(dst_hbm.at[pl.ds(wid * (NCH * CH), NCH * CH)], dst_all)
    plsc.subcore_barrier()

    def sidx(j):
        return src_all.at[pl.ds(j * CH, CH)]

    def didx(j):
        return dst_all.at[pl.ds(j * CH, CH)]

    def gather(j, b):
        pltpu.async_copy(hs_hbm.at[sidx(j)], rows[b], semg[b])

    def gather_wait(j, b):
        pltpu.make_async_copy(hs_hbm.at[sidx(j)], rows[b], semg[b]).wait()

    def scat(j, b):
        pltpu.async_copy(rows[b], acc_sh.at[didx(j)], sems[b], add=True)

    def scat_wait(j, b):
        pltpu.make_async_copy(rows[b], acc_sh.at[didx(j)], sems[b]).wait()

    for b in range(4):
        gather(b, b)

    def body(k, _):
        for i in range(4):
            c = k * 4 + i
            b = i
            gather_wait(c, b)
            scat(c, b)
            br = (i + 2) % 4

            @pl.when(jnp.logical_and(c >= 2, c <= NCH - 3))
            def _(c=c, b=br):
                scat_wait(c - 2, b)
                gather(c + 2, b)

        return 0

    lax.fori_loop(0, NCH // 4, body, 0)
    for c in range(4 * (NCH // 4), NCH):
        gather_wait(c, c % 4)
        scat(c, c % 4)
    for c in range(NCH - 4, NCH):
        scat_wait(c, c % 4)
    plsc.subcore_barrier()

    @pl.when(cid == 0)
    def _():
        pltpu.sync_copy(acc_sh.at[row0], out0_hbm.at[row0])

    @pl.when(cid == 1)
    def _():
        pltpu.sync_copy(acc_sh.at[row0], out1_hbm.at[row0])


_BN = 1000


def _linear_body(x_ref, w_ref, d0_ref, d1_ref, hs_ref, dis_ref):
    deg = d0_ref[...][:, 0:1] + d1_ref[...][:, 0:1] + 1.0
    dis = lax.rsqrt(deg)
    h = jnp.dot(x_ref[...], w_ref[...], preferred_element_type=jnp.float32)
    hs_ref[...] = h * dis
    dis_ref[...] = dis


def _linear(x, W, d0, d1):
    return pl.pallas_call(
        _linear_body,
        grid=(N_NODES // _BN,),
        in_specs=[
            pl.BlockSpec((_BN, D_IN), lambda i: (i, 0)),
            pl.BlockSpec((D_IN, D_OUT), lambda i: (0, 0)),
            pl.BlockSpec((_BN, DEG_W), lambda i: (i, 0)),
            pl.BlockSpec((_BN, DEG_W), lambda i: (i, 0)),
        ],
        out_specs=[
            pl.BlockSpec((_BN, D_OUT), lambda i: (i, 0)),
            pl.BlockSpec((_BN, 1), lambda i: (i, 0)),
        ],
        out_shape=[
            jax.ShapeDtypeStruct((N_NODES, D_OUT), jnp.float32),
            jax.ShapeDtypeStruct((N_NODES, 1), jnp.float32),
        ],
    )(x, W, d0, d1)


def _combine_body(s0_ref, s1_ref, dis_ref, b_ref, out_ref):
    out_ref[...] = dis_ref[...] * (s0_ref[...] + s1_ref[...]) + b_ref[...]


def _combine(s0, s1, dis, b2):
    return pl.pallas_call(
        _combine_body,
        grid=(N_NODES // _BN,),
        in_specs=[
            pl.BlockSpec((_BN, D_OUT), lambda i: (i, 0)),
            pl.BlockSpec((_BN, D_OUT), lambda i: (i, 0)),
            pl.BlockSpec((_BN, 1), lambda i: (i, 0)),
            pl.BlockSpec((1, D_OUT), lambda i: (0, 0)),
        ],
        out_specs=pl.BlockSpec((_BN, D_OUT), lambda i: (i, 0)),
        out_shape=jax.ShapeDtypeStruct((N_NODES, D_OUT), jnp.float32),
    )(s0, s1, dis, b2)


_BE = 320000


def _edge_body(e_ref, src_ref, dst_ref):
    src_ref[...] = jnp.reshape(e_ref[0:1, :], (_BE,))
    dst_ref[...] = jnp.reshape(e_ref[1:2, :], (_BE,))


def _edge_prep(edge_index):
    n_e = edge_index.shape[1]
    return pl.pallas_call(
        _edge_body,
        grid=(n_e // _BE,),
        in_specs=[
            pl.BlockSpec((2, _BE), lambda i: (0, i)),
        ],
        out_specs=[
            pl.BlockSpec((_BE,), lambda i: (i,)),
            pl.BlockSpec((_BE,), lambda i: (i,)),
        ],
        out_shape=[
            jax.ShapeDtypeStruct((n_e,), jnp.int32),
            jax.ShapeDtypeStruct((n_e,), jnp.int32),
        ],
    )(edge_index)


def kernel(x, edge_index, W, b):
    src, dst = _edge_prep(edge_index)
    z_deg = jnp.zeros((RPT, DEG_W), jnp.float32)
    ones = jnp.ones((CH, DEG_W), jnp.float32)
    z_acc = jnp.zeros((RPT, D_OUT), jnp.float32)

    d0, d1 = _deg_kernel(dst, z_deg, ones)
    hs, dis = _linear(x, W, d0, d1)
    s0, s1 = _scatter_kernel(hs, src, dst, z_acc)
    return _combine(s0, s1, dis, jnp.reshape(b, (1, D_OUT)))


# speedup vs baseline: 57.4238x; 1.0570x over previous
import functools

import jax
import jax.numpy as jnp
from jax import lax
from jax.experimental import pallas as pl
from jax.experimental.pallas import tpu as pltpu
from jax.experimental.pallas import tpu_sc as plsc

N_NODES = 10000
D_IN = 128
D_OUT = 64

NC = 2
NS = 16
NW = NC * NS
NPAD = 10240
RPT = NPAD // NS
LASTR = N_NODES - 15 * RPT
CH = 200
NCH = 50
N_EDGES = NW * NCH * CH
DEG_W = 8

_mesh = plsc.VectorSubcoreMesh(core_axis_name="c", subcore_axis_name="s")
_sc_params = pltpu.CompilerParams(use_tc_tiling_on_sc=False)


@functools.partial(
    pl.kernel,
    out_type=[jax.ShapeDtypeStruct((NPAD, DEG_W), jnp.float32)] * NC,
    mesh=_mesh,
    compiler_params=_sc_params,
    scratch_types=[
        pltpu.VMEM_SHARED((NPAD, DEG_W), jnp.float32),
        pltpu.VMEM((NCH * CH,), jnp.int32),
        pltpu.VMEM((CH, DEG_W), jnp.float32),
        pltpu.SemaphoreType.DMA,
    ],
)
def _deg_kernel(dst_hbm, zeros_hbm, ones_hbm, out0_hbm, out1_hbm,
                acc_sh, dst_all, ones_v, sem):
    cid = lax.axis_index("c")
    sid = lax.axis_index("s")
    wid = sid * NC + cid
    row0 = pl.ds(sid * RPT, RPT)
    pltpu.sync_copy(zeros_hbm, acc_sh.at[row0])
    pltpu.sync_copy(ones_hbm, ones_v)
    pltpu.sync_copy(dst_hbm.at[pl.ds(wid * (NCH * CH), NCH * CH)], dst_all)
    plsc.subcore_barrier()

    GRP = 5
    NG = NCH // GRP

    def idx(j):
        return dst_all.at[pl.ds(j * CH, CH)]

    def fire(g):
        for i in range(GRP):
            pltpu.async_copy(ones_v, acc_sh.at[idx(g * GRP + i)], sem, add=True)

    def drain(g):
        for i in range(GRP):
            pltpu.make_async_copy(ones_v, acc_sh.at[idx(g * GRP + i)],
                                  sem).wait()

    fire(0)

    def body(g, _):
        fire(g)
        drain(g - 1)
        return 0

    lax.fori_loop(1, NG, body, 0)
    drain(NG - 1)
    plsc.subcore_barrier()

    @pl.when(cid == 0)
    def _():
        pltpu.sync_copy(acc_sh.at[row0], out0_hbm.at[row0])

    @pl.when(cid == 1)
    def _():
        pltpu.sync_copy(acc_sh.at[row0], out1_hbm.at[row0])


@functools.partial(
    pl.kernel,
    out_type=[jax.ShapeDtypeStruct((NPAD, D_OUT), jnp.bfloat16)] * (2 * NC),
    mesh=_mesh,
    compiler_params=_sc_params,
    scratch_types=[
        [pltpu.VMEM_SHARED((NPAD, D_OUT), jnp.bfloat16)] * 2,
        pltpu.VMEM((NCH * CH,), jnp.int32),
        pltpu.VMEM((NCH * CH,), jnp.int32),
        [pltpu.VMEM((CH, D_OUT), jnp.bfloat16)] * 4,
        [pltpu.SemaphoreType.DMA] * 4,
        [pltpu.SemaphoreType.DMA] * 4,
    ],
)
def _scatter_kernel(hs_hbm, src_hbm, dst_hbm, zeros_hbm,
                    outa0_hbm, outb0_hbm, outa1_hbm, outb1_hbm,
                    accs, src_all, dst_all, rows, semg, sems):
    cid = lax.axis_index("c")
    sid = lax.axis_index("s")
    wid = sid * NC + cid
    row0 = pl.ds(sid * RPT, RPT)
    rowl = pl.ds(15 * RPT, LASTR)

    @pl.when(jnp.logical_and(cid == 0, sid < 15))
    def _():
        pltpu.sync_copy(hs_hbm.at[row0], accs[0].at[row0])

    @pl.when(jnp.logical_and(cid == 0, sid == 15))
    def _():
        pltpu.sync_copy(hs_hbm.at[rowl], accs[0].at[rowl])

    @pl.when(jnp.logical_and(cid == 1, sid < 15))
    def _():
        pltpu.sync_copy(zeros_hbm, accs[0].at[row0])

    @pl.when(jnp.logical_and(cid == 1, sid == 15))
    def _():
        pltpu.sync_copy(zeros_hbm.at[pl.ds(0, LASTR)], accs[0].at[rowl])

    @pl.when(sid < 15)
    def _():
        pltpu.sync_copy(zeros_hbm, accs[1].at[row0])

    @pl.when(sid == 15)
    def _():
        pltpu.sync_copy(zeros_hbm.at[pl.ds(0, LASTR)], accs[1].at[rowl])

    pltpu.sync_copy(src_hbm.at[pl.ds(wid * (NCH * CH), NCH * CH)], src_all)
    pltpu.sync_copy(dst_hbm.at[pl.ds(wid * (NCH * CH), NCH * CH)], dst_all)
    plsc.subcore_barrier()

    def sidx(j):
        return src_all.at[pl.ds(j * CH, CH)]

    def didx(j):
        return dst_all.at[pl.ds(j * CH, CH)]

    def gather(j, b):
        pltpu.async_copy(hs_hbm.at[sidx(j)], rows[b], semg[b])

    def gather_wait(j, b):
        pltpu.make_async_copy(hs_hbm.at[sidx(j)], rows[b], semg[b]).wait()

    def scat(j, b):
        pltpu.async_copy(rows[b], accs[b % 2].at[didx(j)], sems[b], add=True)

    def scat_wait(j, b):
        pltpu.make_async_copy(rows[b], accs[b % 2].at[didx(j)], sems[b]).wait()

    for b in range(4):
        gather(b, b)

    def body(k, _):
        for i in range(4):
            c = k * 4 + i
            b = i
            gather_wait(c, b)
            scat(c, b)
            br = (i + 2) % 4

            @pl.when(jnp.logical_and(c >= 2, c <= NCH - 3))
            def _(c=c, b=br):
                scat_wait(c - 2, b)
                gather(c + 2, b)

        return 0

    lax.fori_loop(0, NCH // 4, body, 0)
    for c in range(4 * (NCH // 4), NCH):
        gather_wait(c, c % 4)
        scat(c, c % 4)
    for c in range(NCH - 4, NCH):
        scat_wait(c, c % 4)
    plsc.subcore_barrier()

    @pl.when(cid == 0)
    def _():
        pltpu.sync_copy(accs[0].at[row0], outa0_hbm.at[row0])
        pltpu.sync_copy(accs[1].at[row0], outb0_hbm.at[row0])

    @pl.when(cid == 1)
    def _():
        pltpu.sync_copy(accs[0].at[row0], outa1_hbm.at[row0])
        pltpu.sync_copy(accs[1].at[row0], outb1_hbm.at[row0])


_BN = 1000


def _linear_body(x_ref, w_ref, d0_ref, d1_ref, hs_ref, dis_ref):
    deg = d0_ref[...][:, 0:1] + d1_ref[...][:, 0:1] + 1.0
    dis = lax.rsqrt(deg)
    h = jnp.dot(x_ref[...], w_ref[...], preferred_element_type=jnp.float32)
    hs_ref[...] = (h * dis).astype(jnp.bfloat16)
    dis_ref[...] = dis


def _linear(x, W, d0, d1):
    return pl.pallas_call(
        _linear_body,
        grid=(N_NODES // _BN,),
        in_specs=[
            pl.BlockSpec((_BN, D_IN), lambda i: (i, 0)),
            pl.BlockSpec((D_IN, D_OUT), lambda i: (0, 0)),
            pl.BlockSpec((_BN, DEG_W), lambda i: (i, 0)),
            pl.BlockSpec((_BN, DEG_W), lambda i: (i, 0)),
        ],
        out_specs=[
            pl.BlockSpec((_BN, D_OUT), lambda i: (i, 0)),
            pl.BlockSpec((_BN, 1), lambda i: (i, 0)),
        ],
        out_shape=[
            jax.ShapeDtypeStruct((N_NODES, D_OUT), jnp.bfloat16),
            jax.ShapeDtypeStruct((N_NODES, 1), jnp.float32),
        ],
    )(x, W, d0, d1)


def _combine_body(s0_ref, s1_ref, s2_ref, s3_ref, dis_ref, b_ref, out_ref):
    s = (s0_ref[...].astype(jnp.float32) + s1_ref[...].astype(jnp.float32)
         + s2_ref[...].astype(jnp.float32) + s3_ref[...].astype(jnp.float32))
    out_ref[...] = dis_ref[...] * s + b_ref[...]


def _combine(parts, dis, b2):
    return pl.pallas_call(
        _combine_body,
        grid=(N_NODES // _BN,),
        in_specs=[pl.BlockSpec((_BN, D_OUT), lambda i: (i, 0))] * 4 + [
            pl.BlockSpec((_BN, 1), lambda i: (i, 0)),
            pl.BlockSpec((1, D_OUT), lambda i: (0, 0)),
        ],
        out_specs=pl.BlockSpec((_BN, D_OUT), lambda i: (i, 0)),
        out_shape=jax.ShapeDtypeStruct((N_NODES, D_OUT), jnp.float32),
    )(*parts, dis, b2)


_BE = 320000


def _edge_body(e_ref, src_ref, dst_ref):
    src_ref[...] = jnp.reshape(e_ref[0:1, :], (_BE,))
    dst_ref[...] = jnp.reshape(e_ref[1:2, :], (_BE,))


def _edge_prep(edge_index):
    n_e = edge_index.shape[1]
    return pl.pallas_call(
        _edge_body,
        grid=(n_e // _BE,),
        in_specs=[
            pl.BlockSpec((2, _BE), lambda i: (0, i)),
        ],
        out_specs=[
            pl.BlockSpec((_BE,), lambda i: (i,)),
            pl.BlockSpec((_BE,), lambda i: (i,)),
        ],
        out_shape=[
            jax.ShapeDtypeStruct((n_e,), jnp.int32),
            jax.ShapeDtypeStruct((n_e,), jnp.int32),
        ],
    )(edge_index)


def kernel(x, edge_index, W, b):
    src, dst = _edge_prep(edge_index)
    z_deg = jnp.zeros((RPT, DEG_W), jnp.float32)
    ones = jnp.ones((CH, DEG_W), jnp.float32)
    z_acc = jnp.zeros((RPT, D_OUT), jnp.bfloat16)

    d0, d1 = _deg_kernel(dst, z_deg, ones)
    hs, dis = _linear(x, W, d0, d1)
    parts = _scatter_kernel(hs, src, dst, z_acc)
    return _combine(parts, dis, jnp.reshape(b, (1, D_OUT)))


# speedup vs baseline: 59.2292x; 1.0314x over previous
import functools

import jax
import jax.numpy as jnp
from jax import lax
from jax.experimental import pallas as pl
from jax.experimental.pallas import tpu as pltpu
from jax.experimental.pallas import tpu_sc as plsc

N_NODES = 10000
D_IN = 128
D_OUT = 64

NC = 2
NS = 16
NW = NC * NS
NPAD = 10240
RPT = NPAD // NS
LASTR = N_NODES - 15 * RPT
CH = 200
NCH = 50
N_EDGES = NW * NCH * CH
DEG_W = 8

_mesh = plsc.VectorSubcoreMesh(core_axis_name="c", subcore_axis_name="s")
_sc_params = pltpu.CompilerParams(use_tc_tiling_on_sc=False)


@functools.partial(
    pl.kernel,
    out_type=[jax.ShapeDtypeStruct((NPAD, DEG_W), jnp.float32)] * NC,
    mesh=_mesh,
    compiler_params=_sc_params,
    scratch_types=[
        pltpu.VMEM_SHARED((NPAD, DEG_W), jnp.float32),
        pltpu.VMEM((NCH * CH,), jnp.int32),
        pltpu.VMEM((CH, DEG_W), jnp.float32),
        pltpu.SemaphoreType.DMA,
    ],
)
def _deg_kernel(dst_hbm, zeros_hbm, ones_hbm, out0_hbm, out1_hbm,
                acc_sh, dst_all, ones_v, sem):
    cid = lax.axis_index("c")
    sid = lax.axis_index("s")
    wid = sid * NC + cid
    row0 = pl.ds(sid * RPT, RPT)
    pltpu.sync_copy(zeros_hbm, acc_sh.at[row0])
    pltpu.sync_copy(ones_hbm, ones_v)
    pltpu.sync_copy(dst_hbm.at[pl.ds(wid * (NCH * CH), NCH * CH)], dst_all)
    plsc.subcore_barrier()

    GRP = 5
    NG = NCH // GRP

    def idx(j):
        return dst_all.at[pl.ds(j * CH, CH)]

    def fire(g):
        for i in range(GRP):
            pltpu.async_copy(ones_v, acc_sh.at[idx(g * GRP + i)], sem, add=True)

    def drain(g):
        for i in range(GRP):
            pltpu.make_async_copy(ones_v, acc_sh.at[idx(g * GRP + i)],
                                  sem).wait()

    fire(0)

    def body(g, _):
        fire(g)
        drain(g - 1)
        return 0

    lax.fori_loop(1, NG, body, 0)
    drain(NG - 1)
    plsc.subcore_barrier()

    @pl.when(cid == 0)
    def _():
        pltpu.sync_copy(acc_sh.at[row0], out0_hbm.at[row0])

    @pl.when(cid == 1)
    def _():
        pltpu.sync_copy(acc_sh.at[row0], out1_hbm.at[row0])


@functools.partial(
    pl.kernel,
    out_type=[jax.ShapeDtypeStruct((NPAD, D_OUT), jnp.bfloat16)] * (2 * NC),
    mesh=_mesh,
    compiler_params=_sc_params,
    scratch_types=[
        [pltpu.VMEM_SHARED((NPAD, D_OUT), jnp.bfloat16)] * 2,
        pltpu.VMEM((NCH * CH,), jnp.int32),
        pltpu.VMEM((NCH * CH,), jnp.int32),
        [pltpu.VMEM((CH, D_OUT), jnp.bfloat16)] * 4,
        [pltpu.SemaphoreType.DMA] * 4,
        [pltpu.SemaphoreType.DMA] * 4,
    ],
)
def _scatter_kernel(hs_hbm, src_hbm, dst_hbm, zeros_hbm,
                    outa0_hbm, outb0_hbm, outa1_hbm, outb1_hbm,
                    accs, src_all, dst_all, rows, semg, sems):
    cid = lax.axis_index("c")
    sid = lax.axis_index("s")
    wid = sid * NC + cid
    row0 = pl.ds(sid * RPT, RPT)
    rowl = pl.ds(15 * RPT, LASTR)

    @pl.when(jnp.logical_and(cid == 0, sid < 15))
    def _():
        pltpu.sync_copy(hs_hbm.at[row0], accs[0].at[row0])

    @pl.when(jnp.logical_and(cid == 0, sid == 15))
    def _():
        pltpu.sync_copy(hs_hbm.at[rowl], accs[0].at[rowl])

    @pl.when(jnp.logical_and(cid == 1, sid < 15))
    def _():
        pltpu.sync_copy(zeros_hbm, accs[0].at[row0])

    @pl.when(jnp.logical_and(cid == 1, sid == 15))
    def _():
        pltpu.sync_copy(zeros_hbm.at[pl.ds(0, LASTR)], accs[0].at[rowl])

    @pl.when(sid < 15)
    def _():
        pltpu.sync_copy(zeros_hbm, accs[1].at[row0])

    @pl.when(sid == 15)
    def _():
        pltpu.sync_copy(zeros_hbm.at[pl.ds(0, LASTR)], accs[1].at[rowl])

    pltpu.sync_copy(src_hbm.at[pl.ds(wid * (NCH * CH), NCH * CH)], src_all)
    pltpu.sync_copy(dst_hbm.at[pl.ds(wid * (NCH * CH), NCH * CH)], dst_all)
    plsc.subcore_barrier()

    def sidx(j):
        return src_all.at[pl.ds(j * CH, CH)]

    def didx(j):
        return dst_all.at[pl.ds(j * CH, CH)]

    def gather(j, b):
        pltpu.async_copy(hs_hbm.at[sidx(j)], rows[b], semg[b])

    def gather_wait(j, b):
        pltpu.make_async_copy(hs_hbm.at[sidx(j)], rows[b], semg[b]).wait()

    def scat(j, b):
        pltpu.async_copy(rows[b], accs[b % 2].at[didx(j)], sems[b], add=True)

    def scat_wait(j, b):
        pltpu.make_async_copy(rows[b], accs[b % 2].at[didx(j)], sems[b]).wait()

    for b in range(4):
        gather(b, b)

    def body(k, _):
        for i in range(4):
            c = k * 4 + i
            b = i
            gather_wait(c, b)
            scat(c, b)
            br = (i + 2) % 4

            @pl.when(jnp.logical_and(c >= 2, c <= NCH - 3))
            def _(c=c, b=br):
                scat_wait(c - 2, b)
                gather(c + 2, b)

        return 0

    lax.fori_loop(0, NCH // 4, body, 0)
    for c in range(4 * (NCH // 4), NCH):
        gather_wait(c, c % 4)
        scat(c, c % 4)
    for c in range(NCH - 4, NCH):
        scat_wait(c, c % 4)
    plsc.subcore_barrier()

    @pl.when(cid == 0)
    def _():
        pltpu.sync_copy(accs[0].at[row0], outa0_hbm.at[row0])
        pltpu.sync_copy(accs[1].at[row0], outb0_hbm.at[row0])

    @pl.when(cid == 1)
    def _():
        pltpu.sync_copy(accs[0].at[row0], outa1_hbm.at[row0])
        pltpu.sync_copy(accs[1].at[row0], outb1_hbm.at[row0])


_BN = 2000


def _linear_body(x_ref, w_ref, d0_ref, d1_ref, hs_ref, dis_ref):
    deg = d0_ref[...][:, 0:1] + d1_ref[...][:, 0:1] + 1.0
    dis = lax.rsqrt(deg)
    h = jnp.dot(x_ref[...], w_ref[...], preferred_element_type=jnp.float32)
    hs_ref[...] = (h * dis).astype(jnp.bfloat16)
    dis_ref[...] = dis


def _linear(x, W, d0, d1):
    return pl.pallas_call(
        _linear_body,
        grid=(N_NODES // _BN,),
        compiler_params=pltpu.CompilerParams(
            dimension_semantics=("arbitrary",)),
        in_specs=[
            pl.BlockSpec((_BN, D_IN), lambda i: (i, 0)),
            pl.BlockSpec((D_IN, D_OUT), lambda i: (0, 0)),
            pl.BlockSpec((_BN, DEG_W), lambda i: (i, 0)),
            pl.BlockSpec((_BN, DEG_W), lambda i: (i, 0)),
        ],
        out_specs=[
            pl.BlockSpec((_BN, D_OUT), lambda i: (i, 0)),
            pl.BlockSpec((_BN, 1), lambda i: (i, 0)),
        ],
        out_shape=[
            jax.ShapeDtypeStruct((N_NODES, D_OUT), jnp.bfloat16),
            jax.ShapeDtypeStruct((N_NODES, 1), jnp.float32),
        ],
    )(x, W, d0, d1)


def _combine_body(s0_ref, s1_ref, s2_ref, s3_ref, dis_ref, b_ref, out_ref):
    s = (s0_ref[...].astype(jnp.float32) + s1_ref[...].astype(jnp.float32)
         + s2_ref[...].astype(jnp.float32) + s3_ref[...].astype(jnp.float32))
    out_ref[...] = dis_ref[...] * s + b_ref[...]


def _combine(parts, dis, b2):
    return pl.pallas_call(
        _combine_body,
        grid=(N_NODES // _BN,),
        compiler_params=pltpu.CompilerParams(
            dimension_semantics=("arbitrary",)),
        in_specs=[pl.BlockSpec((_BN, D_OUT), lambda i: (i, 0))] * 4 + [
            pl.BlockSpec((_BN, 1), lambda i: (i, 0)),
            pl.BlockSpec((1, D_OUT), lambda i: (0, 0)),
        ],
        out_specs=pl.BlockSpec((_BN, D_OUT), lambda i: (i, 0)),
        out_shape=jax.ShapeDtypeStruct((N_NODES, D_OUT), jnp.float32),
    )(*parts, dis, b2)


_BE = 320000


def _edge_body(e_ref, src_ref, dst_ref):
    src_ref[...] = jnp.reshape(e_ref[0:1, :], (_BE,))
    dst_ref[...] = jnp.reshape(e_ref[1:2, :], (_BE,))


def _edge_prep(edge_index):
    n_e = edge_index.shape[1]
    return pl.pallas_call(
        _edge_body,
        grid=(n_e // _BE,),
        in_specs=[
            pl.BlockSpec((2, _BE), lambda i: (0, i)),
        ],
        out_specs=[
            pl.BlockSpec((_BE,), lambda i: (i,)),
            pl.BlockSpec((_BE,), lambda i: (i,)),
        ],
        out_shape=[
            jax.ShapeDtypeStruct((n_e,), jnp.int32),
            jax.ShapeDtypeStruct((n_e,), jnp.int32),
        ],
    )(edge_index)


def kernel(x, edge_index, W, b):
    src, dst = _edge_prep(edge_index)
    z_deg = jnp.zeros((RPT, DEG_W), jnp.float32)
    ones = jnp.ones((CH, DEG_W), jnp.float32)
    z_acc = jnp.zeros((RPT, D_OUT), jnp.bfloat16)

    d0, d1 = _deg_kernel(dst, z_deg, ones)
    hs, dis = _linear(x, W, d0, d1)
    parts = _scatter_kernel(hs, src, dst, z_acc)
    return _combine(parts, dis, jnp.reshape(b, (1, D_OUT)))
